# baseline (speedup 1.0000x reference)
import jax
import jax.numpy as jnp
from jax import lax
from jax.experimental import pallas as pl
from jax.experimental.pallas import tpu as pltpu
from jax.experimental.pallas import tpu_sc as plsc

N = 10000
E = 150000
F_IN = 128
HID = 128
HEADS = 8
DH = 16
C = 16

TW = 144
R = 10240
NS = 16
RPT = R // NS
EPT = 9728
ECH = 64
NCH = EPT // ECH
EPAD = NS * EPT
BLK = 1024

f32 = jnp.float32


def _sc_edge_kernel(tab0, tab1, ad0, ad1, src0, dst0, src1, dst1,
                    out0, out1, acc):
    c = lax.axis_index("c")
    s = lax.axis_index("s")

    def scoped(hs0, hs1, av0, av1, si0, si1, di0, di1,
               gsem0, gsem1, ssem0, ssem1):
        z16 = jnp.zeros((DH,), f32)

        def zrow(r, carry):
            for cc in range(TW // DH):
                hs0[r, pl.ds(cc * DH, DH)] = z16
            return carry

        lax.fori_loop(0, ECH, zrow, 0)
        for k in range(RPT // ECH):
            pltpu.sync_copy(hs0, acc.at[pl.ds(s * RPT + k * ECH, ECH)])
        plsc.subcore_barrier()
        _sc_edge_body(tab0, tab1, ad0, ad1, src0, dst0, src1, dst1,
                      out0, out1, acc, c, s,
                      (hs0, hs1), (av0, av1), (si0, si1), (di0, di1),
                      (gsem0, gsem1), (ssem0, ssem1))

    pl.run_scoped(
        scoped,
        pltpu.VMEM((ECH, TW), f32), pltpu.VMEM((ECH, TW), f32),
        pltpu.VMEM((ECH, DH), f32), pltpu.VMEM((ECH, DH), f32),
        pltpu.VMEM((ECH,), jnp.int32), pltpu.VMEM((ECH,), jnp.int32),
        pltpu.VMEM((ECH,), jnp.int32), pltpu.VMEM((ECH,), jnp.int32),
        pltpu.SemaphoreType.DMA, pltpu.SemaphoreType.DMA,
        pltpu.SemaphoreType.DMA, pltpu.SemaphoreType.DMA,
    )


def _sc_edge_body(tab0, tab1, ad0, ad1, src0, dst0, src1, dst1,
                  out0, out1, acc, c, s, hs, av, si, di, gsem, ssem):
    hs0, hs1 = hs
    di0, di1 = di
    ssem0, ssem1 = ssem

    def run():
        def fetch(i, b):
            base = s * EPT + i * ECH

            @pl.when(c == 0)
            def _():
                pltpu.sync_copy(src0.at[pl.ds(base, ECH)], si[b])
                pltpu.sync_copy(dst0.at[pl.ds(base, ECH)], di[b])
                pltpu.async_copy(tab0.at[si[b]], hs[b], gsem[b])
                pltpu.async_copy(ad0.at[di[b]], av[b], gsem[b])

            @pl.when(c == 1)
            def _():
                pltpu.sync_copy(src1.at[pl.ds(base, ECH)], si[b])
                pltpu.sync_copy(dst1.at[pl.ds(base, ECH)], di[b])
                pltpu.async_copy(tab1.at[si[b]], hs[b], gsem[b])
                pltpu.async_copy(ad1.at[di[b]], av[b], gsem[b])

        def compute(b):
            hs_v = hs[b]
            ad_v = av[b]

            @plsc.parallel_loop(0, ECH, 1, unroll=8)
            def _(e):
                a = hs_v[e, pl.ds(HID, DH)] + ad_v[e, pl.ds(0, DH)]
                a = jnp.where(a > 0.0, a, 0.2 * a)
                ex = jnp.exp(a)
                hs_v[e, pl.ds(HID, DH)] = ex
                for h in range(HEADS):
                    hs_v[e, pl.ds(h * DH, DH)] = hs_v[e, pl.ds(h * DH, DH)] * ex[h]

        fetch(0, 0)

        def body(j, carry):
            for b in (0, 1):
                i = 2 * j + b
                nb = 1 - b

                @pl.when(i + 1 < NCH)
                def _():
                    @pl.when(i >= 1)
                    def _():
                        pltpu.make_async_copy(hs[nb], acc.at[di[nb]], ssem[nb]).wait()
                    fetch(i + 1, nb)

                pltpu.make_async_copy(tab0.at[si[b]], hs[b], gsem[b]).wait()
                pltpu.make_async_copy(ad0.at[di[b]], av[b], gsem[b]).wait()
                compute(b)
                pltpu.async_copy(hs[b], acc.at[di[b]], ssem[b], add=True)
            return carry

        lax.fori_loop(0, NCH // 2, body, 0)
        pltpu.make_async_copy(hs0, acc.at[di0], ssem0).wait()
        pltpu.make_async_copy(hs1, acc.at[di1], ssem1).wait()

    run()

    plsc.subcore_barrier()

    @pl.when(c == 0)
    def _():
        for k in range(RPT // ECH):
            b = s * RPT + k * ECH
            pltpu.sync_copy(acc.at[pl.ds(b, ECH)], out0.at[pl.ds(b, ECH)])

    @pl.when(c == 1)
    def _():
        for k in range(RPT // ECH):
            b = s * RPT + k * ECH
            pltpu.sync_copy(acc.at[pl.ds(b, ECH)], out1.at[pl.ds(b, ECH)])


def _sc_call(tab0, tab1, ad0, ad1, src0, dst0, src1, dst1):
    mesh = plsc.VectorSubcoreMesh(core_axis_name="c", subcore_axis_name="s")
    fn = pl.kernel(
        _sc_edge_kernel,
        out_type=(jax.ShapeDtypeStruct((R, TW), f32),
                  jax.ShapeDtypeStruct((R, TW), f32)),
        mesh=mesh,
        scratch_types=[
            pltpu.VMEM_SHARED((R, TW), f32),
        ],
        compiler_params=pltpu.CompilerParams(use_tc_tiling_on_sc=False),
    )
    return fn(tab0, tab1, ad0, ad1, src0, dst0, src1, dst1)


def _tca_body(xd, xa, xm, wd, bd, wa, ba, wm, bm, asd, asa, atd, ata,
              t0, t1, a0, a1):
    hd = jnp.dot(xd[:], wd[:], preferred_element_type=f32) + bd[:]
    ha = jnp.dot(xa[:], wa[:], preferred_element_type=f32) + ba[:]
    hm = jnp.dot(xm[:], wm[:], preferred_element_type=f32) + bm[:]
    t0[:] = jnp.concatenate([hd, jnp.dot(hd, asd[:], preferred_element_type=f32)], axis=1)
    t1[:] = jnp.concatenate([ha, jnp.dot(ha, asa[:], preferred_element_type=f32)], axis=1)
    a0[:] = jnp.dot(hm, atd[:], preferred_element_type=f32)
    a1[:] = jnp.dot(hm, ata[:], preferred_element_type=f32)


def _tcb1_body(acc0, acc1, wk, bk, rep, agg0, agg1, cs0, cs1):
    g = pl.program_id(0)
    rows = g * BLK + lax.broadcasted_iota(jnp.int32, (BLK, 1), 0)
    mask = rows < N
    for accr, aggr, csr in ((acc0, agg0, cs0), (acc1, agg1, cs1)):
        num = accr[:, 0:HID]
        den = accr[:, HID:HID + HEADS]
        dexp = jnp.dot(den, rep[:], preferred_element_type=f32) + 1e-16
        agg = jnp.maximum(num / dexp, 0.0)
        aggr[:] = agg
        t = jnp.tanh(jnp.dot(agg, wk[:], preferred_element_type=f32) + bk[:])
        t = jnp.where(mask, t, 0.0)
        part = jnp.sum(t, axis=0, keepdims=True)

        @pl.when(g == 0)
        def _():
            csr[:] = part

        @pl.when(g != 0)
        def _():
            csr[:] = csr[:] + part


def _tcb2_body(agg0, agg1, cs0, cs1, q, wl, bl, out):
    s0 = jnp.sum(cs0[:] * q[:]) / float(N)
    s1 = jnp.sum(cs1[:] * q[:]) / float(N)
    m = jnp.maximum(s0, s1)
    e0 = jnp.exp(s0 - m)
    e1 = jnp.exp(s1 - m)
    w0 = e0 / (e0 + e1)
    w1 = e1 / (e0 + e1)
    sem = w0 * agg0[:] + w1 * agg1[:]
    out[:] = jnp.dot(sem, wl[:], preferred_element_type=f32) + bl[:]


def _blockdiag(att):
    eye = jnp.eye(HEADS, dtype=f32)
    m = (att[:, :, None] * eye[:, None, :]).reshape(HID, HEADS)
    return jnp.pad(m, ((0, 0), (0, 16 - HEADS)))


def _pad_rows(x):
    return jnp.pad(x, ((0, R - N), (0, 0)))


def _pad_edges(ei):
    src = jnp.full((EPAD,), N, jnp.int32).at[:E].set(ei[0])
    dst = jnp.full((EPAD,), N, jnp.int32).at[:E].set(ei[1])
    return src, dst


def kernel(x_movie, x_director, x_actor,
           ei_movie_director, ei_director_movie, ei_movie_actor, ei_actor_movie,
           W_proj_movie, b_proj_movie, W_proj_director, b_proj_director,
           W_proj_actor, b_proj_actor,
           att_src_movie_director, att_dst_movie_director,
           att_src_director_movie, att_dst_director_movie,
           att_src_movie_actor, att_dst_movie_actor,
           att_src_actor_movie, att_dst_actor_movie,
           q_sem, W_k, b_k, W_lin, b_lin):
    grid = R // BLK

    tab0, tab1, ad0, ad1 = pl.pallas_call(
        _tca_body,
        grid=(grid,),
        in_specs=[
            pl.BlockSpec((BLK, F_IN), lambda i: (i, 0)),
            pl.BlockSpec((BLK, F_IN), lambda i: (i, 0)),
            pl.BlockSpec((BLK, F_IN), lambda i: (i, 0)),
            pl.BlockSpec((F_IN, HID), lambda i: (0, 0)),
            pl.BlockSpec((1, HID), lambda i: (0, 0)),
            pl.BlockSpec((F_IN, HID), lambda i: (0, 0)),
            pl.BlockSpec((1, HID), lambda i: (0, 0)),
            pl.BlockSpec((F_IN, HID), lambda i: (0, 0)),
            pl.BlockSpec((1, HID), lambda i: (0, 0)),
            pl.BlockSpec((HID, 16), lambda i: (0, 0)),
            pl.BlockSpec((HID, 16), lambda i: (0, 0)),
            pl.BlockSpec((HID, 16), lambda i: (0, 0)),
            pl.BlockSpec((HID, 16), lambda i: (0, 0)),
        ],
        out_specs=[
            pl.BlockSpec((BLK, TW), lambda i: (i, 0)),
            pl.BlockSpec((BLK, TW), lambda i: (i, 0)),
            pl.BlockSpec((BLK, 16), lambda i: (i, 0)),
            pl.BlockSpec((BLK, 16), lambda i: (i, 0)),
        ],
        out_shape=[
            jax.ShapeDtypeStruct((R, TW), f32),
            jax.ShapeDtypeStruct((R, TW), f32),
            jax.ShapeDtypeStruct((R, 16), f32),
            jax.ShapeDtypeStruct((R, 16), f32),
        ],
    )(
        _pad_rows(x_director), _pad_rows(x_actor), _pad_rows(x_movie),
        W_proj_director, b_proj_director.reshape(1, HID),
        W_proj_actor, b_proj_actor.reshape(1, HID),
        W_proj_movie, b_proj_movie.reshape(1, HID),
        _blockdiag(att_src_director_movie), _blockdiag(att_src_actor_movie),
        _blockdiag(att_dst_director_movie), _blockdiag(att_dst_actor_movie),
    )

    src0, dst0 = _pad_edges(ei_director_movie)
    src1, dst1 = _pad_edges(ei_actor_movie)
    acc0, acc1 = _sc_call(tab0, tab1, ad0, ad1, src0, dst0, src1, dst1)

    rep_cols = (jnp.arange(HID, dtype=jnp.int32) // DH)
    rep = (rep_cols[None, :] == jnp.arange(HEADS, dtype=jnp.int32)[:, None]).astype(f32)
    agg0, agg1, cs0, cs1 = pl.pallas_call(
        _tcb1_body,
        grid=(grid,),
        in_specs=[
            pl.BlockSpec((BLK, TW), lambda i: (i, 0)),
            pl.BlockSpec((BLK, TW), lambda i: (i, 0)),
            pl.BlockSpec((HID, HID), lambda i: (0, 0)),
            pl.BlockSpec((1, HID), lambda i: (0, 0)),
            pl.BlockSpec((HEADS, HID), lambda i: (0, 0)),
        ],
        out_specs=[
            pl.BlockSpec((BLK, HID), lambda i: (i, 0)),
            pl.BlockSpec((BLK, HID), lambda i: (i, 0)),
            pl.BlockSpec((1, HID), lambda i: (0, 0)),
            pl.BlockSpec((1, HID), lambda i: (0, 0)),
        ],
        out_shape=[
            jax.ShapeDtypeStruct((R, HID), f32),
            jax.ShapeDtypeStruct((R, HID), f32),
            jax.ShapeDtypeStruct((1, HID), f32),
            jax.ShapeDtypeStruct((1, HID), f32),
        ],
        compiler_params=pltpu.CompilerParams(
            dimension_semantics=("arbitrary",)),
    )(acc0, acc1, W_k, b_k.reshape(1, HID), rep)

    out = pl.pallas_call(
        _tcb2_body,
        grid=(grid,),
        in_specs=[
            pl.BlockSpec((BLK, HID), lambda i: (i, 0)),
            pl.BlockSpec((BLK, HID), lambda i: (i, 0)),
            pl.BlockSpec((1, HID), lambda i: (0, 0)),
            pl.BlockSpec((1, HID), lambda i: (0, 0)),
            pl.BlockSpec((1, HID), lambda i: (0, 0)),
            pl.BlockSpec((HID, C), lambda i: (0, 0)),
            pl.BlockSpec((1, C), lambda i: (0, 0)),
        ],
        out_specs=pl.BlockSpec((BLK, C), lambda i: (i, 0)),
        out_shape=jax.ShapeDtypeStruct((R, C), f32),
    )(agg0, agg1, cs0, cs1, q_sem.reshape(1, HID), W_lin, b_lin.reshape(1, C))

    return out[:N]


# speedup vs baseline: 1.0215x; 1.0215x over previous
import jax
import jax.numpy as jnp
from jax import lax
from jax.experimental import pallas as pl
from jax.experimental.pallas import tpu as pltpu
from jax.experimental.pallas import tpu_sc as plsc

N = 10000
E = 150000
F_IN = 128
HID = 128
HEADS = 8
DH = 16
C = 16

TW = 144
R = 10240
NS = 16
RPT = R // NS
EPT = 9792
ECH = 96
NCH = EPT // ECH
EPAD = NS * EPT
BLK = 1024

f32 = jnp.float32


def _sc_edge_kernel(tab0, tab1, ad0, ad1, src0, dst0, src1, dst1,
                    out0, out1, acc):
    c = lax.axis_index("c")
    s = lax.axis_index("s")

    def scoped(hs0, hs1, av0, av1, si0, si1, di0, di1,
               gsem0, gsem1, ssem0, ssem1):
        z16 = jnp.zeros((DH,), f32)

        def zrow(r, carry):
            for cc in range(TW // DH):
                hs0[r, pl.ds(cc * DH, DH)] = z16
            return carry

        lax.fori_loop(0, ECH, zrow, 0)
        for k in range(RPT // ECH):
            pltpu.sync_copy(hs0, acc.at[pl.ds(s * RPT + k * ECH, ECH)])
        plsc.subcore_barrier()
        _sc_edge_body(tab0, tab1, ad0, ad1, src0, dst0, src1, dst1,
                      out0, out1, acc, c, s,
                      (hs0, hs1), (av0, av1), (si0, si1), (di0, di1),
                      (gsem0, gsem1), (ssem0, ssem1))

    pl.run_scoped(
        scoped,
        pltpu.VMEM((ECH, TW), f32), pltpu.VMEM((ECH, TW), f32),
        pltpu.VMEM((ECH, DH), f32), pltpu.VMEM((ECH, DH), f32),
        pltpu.VMEM((ECH,), jnp.int32), pltpu.VMEM((ECH,), jnp.int32),
        pltpu.VMEM((ECH,), jnp.int32), pltpu.VMEM((ECH,), jnp.int32),
        pltpu.SemaphoreType.DMA, pltpu.SemaphoreType.DMA,
        pltpu.SemaphoreType.DMA, pltpu.SemaphoreType.DMA,
    )


def _sc_edge_body(tab0, tab1, ad0, ad1, src0, dst0, src1, dst1,
                  out0, out1, acc, c, s, hs, av, si, di, gsem, ssem):
    hs0, hs1 = hs
    di0, di1 = di
    ssem0, ssem1 = ssem

    def run():
        def fetch(i, b):
            base = s * EPT + i * ECH

            @pl.when(c == 0)
            def _():
                pltpu.sync_copy(src0.at[pl.ds(base, ECH)], si[b])
                pltpu.sync_copy(dst0.at[pl.ds(base, ECH)], di[b])
                pltpu.async_copy(tab0.at[si[b]], hs[b], gsem[b])
                pltpu.async_copy(ad0.at[di[b]], av[b], gsem[b])

            @pl.when(c == 1)
            def _():
                pltpu.sync_copy(src1.at[pl.ds(base, ECH)], si[b])
                pltpu.sync_copy(dst1.at[pl.ds(base, ECH)], di[b])
                pltpu.async_copy(tab1.at[si[b]], hs[b], gsem[b])
                pltpu.async_copy(ad1.at[di[b]], av[b], gsem[b])

        def compute(b):
            hs_v = hs[b]
            ad_v = av[b]

            @plsc.parallel_loop(0, ECH, 1, unroll=4)
            def _(e):
                a = hs_v[e, pl.ds(HID, DH)] + ad_v[e, pl.ds(0, DH)]
                a = jnp.where(a > 0.0, a, 0.2 * a)
                ex = jnp.exp(a)
                hs_v[e, pl.ds(HID, DH)] = ex
                for h in range(HEADS):
                    hs_v[e, pl.ds(h * DH, DH)] = hs_v[e, pl.ds(h * DH, DH)] * ex[h]

        fetch(0, 0)

        def body(j, carry):
            for b in (0, 1):
                i = 2 * j + b
                nb = 1 - b

                @pl.when(i + 1 < NCH)
                def _():
                    @pl.when(i >= 1)
                    def _():
                        pltpu.make_async_copy(hs[nb], acc.at[di[nb]], ssem[nb]).wait()
                    fetch(i + 1, nb)

                pltpu.make_async_copy(tab0.at[si[b]], hs[b], gsem[b]).wait()
                pltpu.make_async_copy(ad0.at[di[b]], av[b], gsem[b]).wait()
                compute(b)
                pltpu.async_copy(hs[b], acc.at[di[b]], ssem[b], add=True)
            return carry

        lax.fori_loop(0, NCH // 2, body, 0)
        pltpu.make_async_copy(hs0, acc.at[di0], ssem0).wait()
        pltpu.make_async_copy(hs1, acc.at[di1], ssem1).wait()

    run()

    plsc.subcore_barrier()

    @pl.when(c == 0)
    def _():
        for k in range(RPT // ECH):
            b = s * RPT + k * ECH
            pltpu.sync_copy(acc.at[pl.ds(b, ECH)], out0.at[pl.ds(b, ECH)])

    @pl.when(c == 1)
    def _():
        for k in range(RPT // ECH):
            b = s * RPT + k * ECH
            pltpu.sync_copy(acc.at[pl.ds(b, ECH)], out1.at[pl.ds(b, ECH)])


def _sc_call(tab0, tab1, ad0, ad1, src0, dst0, src1, dst1):
    mesh = plsc.VectorSubcoreMesh(core_axis_name="c", subcore_axis_name="s")
    fn = pl.kernel(
        _sc_edge_kernel,
        out_type=(jax.ShapeDtypeStruct((R, TW), f32),
                  jax.ShapeDtypeStruct((R, TW), f32)),
        mesh=mesh,
        scratch_types=[
            pltpu.VMEM_SHARED((R, TW), f32),
        ],
        compiler_params=pltpu.CompilerParams(use_tc_tiling_on_sc=False),
    )
    return fn(tab0, tab1, ad0, ad1, src0, dst0, src1, dst1)


def _tca_body(xd, xa, xm, wd, bd, wa, ba, wm, bm, asd, asa, atd, ata,
              t0, t1, a0, a1):
    hd = jnp.dot(xd[:], wd[:], preferred_element_type=f32) + bd[:]
    ha = jnp.dot(xa[:], wa[:], preferred_element_type=f32) + ba[:]
    hm = jnp.dot(xm[:], wm[:], preferred_element_type=f32) + bm[:]
    t0[:] = jnp.concatenate([hd, jnp.dot(hd, asd[:], preferred_element_type=f32)], axis=1)
    t1[:] = jnp.concatenate([ha, jnp.dot(ha, asa[:], preferred_element_type=f32)], axis=1)
    a0[:] = jnp.dot(hm, atd[:], preferred_element_type=f32)
    a1[:] = jnp.dot(hm, ata[:], preferred_element_type=f32)


def _tcb1_body(acc0, acc1, wk, bk, rep, agg0, agg1, cs0, cs1):
    g = pl.program_id(0)
    rows = g * BLK + lax.broadcasted_iota(jnp.int32, (BLK, 1), 0)
    mask = rows < N
    for accr, aggr, csr in ((acc0, agg0, cs0), (acc1, agg1, cs1)):
        num = accr[:, 0:HID]
        den = accr[:, HID:HID + HEADS]
        dexp = jnp.dot(den, rep[:], preferred_element_type=f32) + 1e-16
        agg = jnp.maximum(num / dexp, 0.0)
        aggr[:] = agg
        t = jnp.tanh(jnp.dot(agg, wk[:], preferred_element_type=f32) + bk[:])
        t = jnp.where(mask, t, 0.0)
        part = jnp.sum(t, axis=0, keepdims=True)

        @pl.when(g == 0)
        def _():
            csr[:] = part

        @pl.when(g != 0)
        def _():
            csr[:] = csr[:] + part


def _tcb2_body(agg0, agg1, cs0, cs1, q, wl, bl, out):
    s0 = jnp.sum(cs0[:] * q[:]) / float(N)
    s1 = jnp.sum(cs1[:] * q[:]) / float(N)
    m = jnp.maximum(s0, s1)
    e0 = jnp.exp(s0 - m)
    e1 = jnp.exp(s1 - m)
    w0 = e0 / (e0 + e1)
    w1 = e1 / (e0 + e1)
    sem = w0 * agg0[:] + w1 * agg1[:]
    out[:] = jnp.dot(sem, wl[:], preferred_element_type=f32) + bl[:]


def _blockdiag(att):
    eye = jnp.eye(HEADS, dtype=f32)
    m = (att[:, :, None] * eye[:, None, :]).reshape(HID, HEADS)
    return jnp.pad(m, ((0, 0), (0, 16 - HEADS)))


def _pad_rows(x):
    return jnp.pad(x, ((0, R - N), (0, 0)))


def _pad_edges(ei):
    src = jnp.full((EPAD,), N, jnp.int32).at[:E].set(ei[0])
    dst = jnp.full((EPAD,), N, jnp.int32).at[:E].set(ei[1])
    return src, dst


def kernel(x_movie, x_director, x_actor,
           ei_movie_director, ei_director_movie, ei_movie_actor, ei_actor_movie,
           W_proj_movie, b_proj_movie, W_proj_director, b_proj_director,
           W_proj_actor, b_proj_actor,
           att_src_movie_director, att_dst_movie_director,
           att_src_director_movie, att_dst_director_movie,
           att_src_movie_actor, att_dst_movie_actor,
           att_src_actor_movie, att_dst_actor_movie,
           q_sem, W_k, b_k, W_lin, b_lin):
    grid = R // BLK

    tab0, tab1, ad0, ad1 = pl.pallas_call(
        _tca_body,
        grid=(grid,),
        in_specs=[
            pl.BlockSpec((BLK, F_IN), lambda i: (i, 0)),
            pl.BlockSpec((BLK, F_IN), lambda i: (i, 0)),
            pl.BlockSpec((BLK, F_IN), lambda i: (i, 0)),
            pl.BlockSpec((F_IN, HID), lambda i: (0, 0)),
            pl.BlockSpec((1, HID), lambda i: (0, 0)),
            pl.BlockSpec((F_IN, HID), lambda i: (0, 0)),
            pl.BlockSpec((1, HID), lambda i: (0, 0)),
            pl.BlockSpec((F_IN, HID), lambda i: (0, 0)),
            pl.BlockSpec((1, HID), lambda i: (0, 0)),
            pl.BlockSpec((HID, 16), lambda i: (0, 0)),
            pl.BlockSpec((HID, 16), lambda i: (0, 0)),
            pl.BlockSpec((HID, 16), lambda i: (0, 0)),
            pl.BlockSpec((HID, 16), lambda i: (0, 0)),
        ],
        out_specs=[
            pl.BlockSpec((BLK, TW), lambda i: (i, 0)),
            pl.BlockSpec((BLK, TW), lambda i: (i, 0)),
            pl.BlockSpec((BLK, 16), lambda i: (i, 0)),
            pl.BlockSpec((BLK, 16), lambda i: (i, 0)),
        ],
        out_shape=[
            jax.ShapeDtypeStruct((R, TW), f32),
            jax.ShapeDtypeStruct((R, TW), f32),
            jax.ShapeDtypeStruct((R, 16), f32),
            jax.ShapeDtypeStruct((R, 16), f32),
        ],
    )(
        _pad_rows(x_director), _pad_rows(x_actor), _pad_rows(x_movie),
        W_proj_director, b_proj_director.reshape(1, HID),
        W_proj_actor, b_proj_actor.reshape(1, HID),
        W_proj_movie, b_proj_movie.reshape(1, HID),
        _blockdiag(att_src_director_movie), _blockdiag(att_src_actor_movie),
        _blockdiag(att_dst_director_movie), _blockdiag(att_dst_actor_movie),
    )

    src0, dst0 = _pad_edges(ei_director_movie)
    src1, dst1 = _pad_edges(ei_actor_movie)
    acc0, acc1 = _sc_call(tab0, tab1, ad0, ad1, src0, dst0, src1, dst1)

    rep_cols = (jnp.arange(HID, dtype=jnp.int32) // DH)
    rep = (rep_cols[None, :] == jnp.arange(HEADS, dtype=jnp.int32)[:, None]).astype(f32)
    agg0, agg1, cs0, cs1 = pl.pallas_call(
        _tcb1_body,
        grid=(grid,),
        in_specs=[
            pl.BlockSpec((BLK, TW), lambda i: (i, 0)),
            pl.BlockSpec((BLK, TW), lambda i: (i, 0)),
            pl.BlockSpec((HID, HID), lambda i: (0, 0)),
            pl.BlockSpec((1, HID), lambda i: (0, 0)),
            pl.BlockSpec((HEADS, HID), lambda i: (0, 0)),
        ],
        out_specs=[
            pl.BlockSpec((BLK, HID), lambda i: (i, 0)),
            pl.BlockSpec((BLK, HID), lambda i: (i, 0)),
            pl.BlockSpec((1, HID), lambda i: (0, 0)),
            pl.BlockSpec((1, HID), lambda i: (0, 0)),
        ],
        out_shape=[
            jax.ShapeDtypeStruct((R, HID), f32),
            jax.ShapeDtypeStruct((R, HID), f32),
            jax.ShapeDtypeStruct((1, HID), f32),
            jax.ShapeDtypeStruct((1, HID), f32),
        ],
        compiler_params=pltpu.CompilerParams(
            dimension_semantics=("arbitrary",)),
    )(acc0, acc1, W_k, b_k.reshape(1, HID), rep)

    out = pl.pallas_call(
        _tcb2_body,
        grid=(grid,),
        in_specs=[
            pl.BlockSpec((BLK, HID), lambda i: (i, 0)),
            pl.BlockSpec((BLK, HID), lambda i: (i, 0)),
            pl.BlockSpec((1, HID), lambda i: (0, 0)),
            pl.BlockSpec((1, HID), lambda i: (0, 0)),
            pl.BlockSpec((1, HID), lambda i: (0, 0)),
            pl.BlockSpec((HID, C), lambda i: (0, 0)),
            pl.BlockSpec((1, C), lambda i: (0, 0)),
        ],
        out_specs=pl.BlockSpec((BLK, C), lambda i: (i, 0)),
        out_shape=jax.ShapeDtypeStruct((R, C), f32),
    )(agg0, agg1, cs0, cs1, q_sem.reshape(1, HID), W_lin, b_lin.reshape(1, C))

    return out[:N]


# speedup vs baseline: 1.8091x; 1.7711x over previous
import jax
import jax.numpy as jnp
from jax import lax
from jax.experimental import pallas as pl
from jax.experimental.pallas import tpu as pltpu
from jax.experimental.pallas import tpu_sc as plsc

N = 10000
E = 150000
F_IN = 128
HID = 128
HEADS = 8
DH = 16
C = 16

TW = 144
R = 10240
NS = 16
RPT = R // NS
EPT = 9472
ECH = 64
NCH = EPT // ECH
EPAD = NS * EPT
BLK = 1024

f32 = jnp.float32


def _sc_edge_kernel(tab0, tab1, ad0, ad1, ep0, ep1,
                    out0, out1, acc):
    c = lax.axis_index("c")
    s = lax.axis_index("s")

    def scoped(hs0, hs1, av0, av1, pk_all, si0, si1, di0, di1,
               gsem0, gsem1, ssem0, ssem1):
        z16 = jnp.zeros((DH,), f32)

        def zrow(r, carry):
            for cc in range(TW // DH):
                hs0[r, pl.ds(cc * DH, DH)] = z16
            return carry

        lax.fori_loop(0, ECH, zrow, 0)
        for k in range(RPT // ECH):
            pltpu.sync_copy(hs0, acc.at[pl.ds(s * RPT + k * ECH, ECH)])
        plsc.subcore_barrier()
        _sc_edge_body(tab0, tab1, ad0, ad1, ep0, ep1,
                      out0, out1, acc, c, s,
                      (hs0, hs1), (av0, av1), pk_all,
                      (si0, si1), (di0, di1),
                      (gsem0, gsem1), (ssem0, ssem1))

    pl.run_scoped(
        scoped,
        pltpu.VMEM((ECH, TW), f32), pltpu.VMEM((ECH, TW), f32),
        pltpu.VMEM((ECH, DH), f32), pltpu.VMEM((ECH, DH), f32),
        pltpu.VMEM((NCH, ECH), jnp.int32),
        pltpu.VMEM((ECH,), jnp.int32), pltpu.VMEM((ECH,), jnp.int32),
        pltpu.VMEM((ECH,), jnp.int32), pltpu.VMEM((ECH,), jnp.int32),
        pltpu.SemaphoreType.DMA, pltpu.SemaphoreType.DMA,
        pltpu.SemaphoreType.DMA, pltpu.SemaphoreType.DMA,
    )


def _sc_edge_body(tab0, tab1, ad0, ad1, ep0, ep1,
                  out0, out1, acc, c, s, hs, av, pk_all, si, di, gsem, ssem):
    hs0, hs1 = hs
    di0, di1 = di
    ssem0, ssem1 = ssem

    def run():
        @pl.when(c == 0)
        def _():
            pltpu.sync_copy(ep0.at[pl.ds(s * NCH, NCH)], pk_all)

        @pl.when(c == 1)
        def _():
            pltpu.sync_copy(ep1.at[pl.ds(s * NCH, NCH)], pk_all)

        def fetch(i, b):
            for l in range(ECH // DH):
                v = pk_all[i, pl.ds(l * DH, DH)]
                si[b][pl.ds(l * DH, DH)] = lax.bitwise_and(v, 0xFFFF)
                di[b][pl.ds(l * DH, DH)] = lax.shift_right_logical(v, 16)

            @pl.when(c == 0)
            def _():
                pltpu.async_copy(tab0.at[si[b]], hs[b], gsem[b])
                pltpu.async_copy(ad0.at[di[b]], av[b], gsem[b])

            @pl.when(c == 1)
            def _():
                pltpu.async_copy(tab1.at[si[b]], hs[b], gsem[b])
                pltpu.async_copy(ad1.at[di[b]], av[b], gsem[b])

        def compute(b):
            hs_v = hs[b]
            ad_v = av[b]

            @plsc.parallel_loop(0, ECH, 1, unroll=4)
            def _(e):
                a = hs_v[e, pl.ds(HID, DH)] + ad_v[e, pl.ds(0, DH)]
                a = jnp.where(a > 0.0, a, 0.2 * a)
                ex = jnp.exp(a)
                hs_v[e, pl.ds(HID, DH)] = ex
                for h in range(HEADS):
                    hs_v[e, pl.ds(h * DH, DH)] = hs_v[e, pl.ds(h * DH, DH)] * ex[h]

        fetch(0, 0)

        def body(j, carry):
            for b in (0, 1):
                i = 2 * j + b
                nb = 1 - b

                @pl.when(i + 1 < NCH)
                def _():
                    @pl.when(i >= 1)
                    def _():
                        pltpu.make_async_copy(hs[nb], acc.at[di[nb]], ssem[nb]).wait()
                    fetch(i + 1, nb)

                pltpu.make_async_copy(tab0.at[si[b]], hs[b], gsem[b]).wait()
                pltpu.make_async_copy(ad0.at[di[b]], av[b], gsem[b]).wait()
                compute(b)
                pltpu.async_copy(hs[b], acc.at[di[b]], ssem[b], add=True)
            return carry

        lax.fori_loop(0, NCH // 2, body, 0)
        pltpu.make_async_copy(hs0, acc.at[di0], ssem0).wait()
        pltpu.make_async_copy(hs1, acc.at[di1], ssem1).wait()

    run()

    plsc.subcore_barrier()

    @pl.when(c == 0)
    def _():
        for k in range(RPT // ECH):
            b = s * RPT + k * ECH
            pltpu.sync_copy(acc.at[pl.ds(b, ECH)], out0.at[pl.ds(b, ECH)])

    @pl.when(c == 1)
    def _():
        for k in range(RPT // ECH):
            b = s * RPT + k * ECH
            pltpu.sync_copy(acc.at[pl.ds(b, ECH)], out1.at[pl.ds(b, ECH)])


def _sc_call(tab0, tab1, ad0, ad1, ep0, ep1):
    mesh = plsc.VectorSubcoreMesh(core_axis_name="c", subcore_axis_name="s")
    fn = pl.kernel(
        _sc_edge_kernel,
        out_type=(jax.ShapeDtypeStruct((R, TW), f32),
                  jax.ShapeDtypeStruct((R, TW), f32)),
        mesh=mesh,
        scratch_types=[
            pltpu.VMEM_SHARED((R, TW), f32),
        ],
        compiler_params=pltpu.CompilerParams(use_tc_tiling_on_sc=False),
    )
    return fn(tab0, tab1, ad0, ad1, ep0, ep1)


def _tca_body(xd, xa, xm, wd, bd, wa, ba, wm, bm, asd, asa, atd, ata,
              t0, t1, a0, a1):
    hd = jnp.dot(xd[:], wd[:], preferred_element_type=f32) + bd[:]
    ha = jnp.dot(xa[:], wa[:], preferred_element_type=f32) + ba[:]
    hm = jnp.dot(xm[:], wm[:], preferred_element_type=f32) + bm[:]
    t0[:] = jnp.concatenate([hd, jnp.dot(hd, asd[:], preferred_element_type=f32)], axis=1)
    t1[:] = jnp.concatenate([ha, jnp.dot(ha, asa[:], preferred_element_type=f32)], axis=1)
    a0[:] = jnp.dot(hm, atd[:], preferred_element_type=f32)
    a1[:] = jnp.dot(hm, ata[:], preferred_element_type=f32)


def _tcb1_body(acc0, acc1, wk, bk, rep, agg0, agg1, cs0, cs1):
    g = pl.program_id(0)
    rows = g * BLK + lax.broadcasted_iota(jnp.int32, (BLK, 1), 0)
    mask = rows < N
    for accr, aggr, csr in ((acc0, agg0, cs0), (acc1, agg1, cs1)):
        num = accr[:, 0:HID]
        den = accr[:, HID:HID + HEADS]
        dexp = jnp.dot(den, rep[:], preferred_element_type=f32) + 1e-16
        agg = jnp.maximum(num / dexp, 0.0)
        aggr[:] = agg
        t = jnp.tanh(jnp.dot(agg, wk[:], preferred_element_type=f32) + bk[:])
        t = jnp.where(mask, t, 0.0)
        part = jnp.sum(t, axis=0, keepdims=True)

        @pl.when(g == 0)
        def _():
            csr[:] = part

        @pl.when(g != 0)
        def _():
            csr[:] = csr[:] + part


def _tcb2_body(agg0, agg1, cs0, cs1, q, wl, bl, out):
    s0 = jnp.sum(cs0[:] * q[:]) / float(N)
    s1 = jnp.sum(cs1[:] * q[:]) / float(N)
    m = jnp.maximum(s0, s1)
    e0 = jnp.exp(s0 - m)
    e1 = jnp.exp(s1 - m)
    w0 = e0 / (e0 + e1)
    w1 = e1 / (e0 + e1)
    sem = w0 * agg0[:] + w1 * agg1[:]
    out[:] = jnp.dot(sem, wl[:], preferred_element_type=f32) + bl[:]


def _blockdiag(att):
    eye = jnp.eye(HEADS, dtype=f32)
    m = (att[:, :, None] * eye[:, None, :]).reshape(HID, HEADS)
    return jnp.pad(m, ((0, 0), (0, 16 - HEADS)))


def _pad_rows(x):
    return jnp.pad(x, ((0, R - N), (0, 0)))


def _pad_edges(ei):
    pad = jnp.int32(N | (N << 16))
    packed = jnp.full((EPAD,), pad, jnp.int32).at[:E].set(ei[0] | (ei[1] << 16))
    return packed.reshape(EPAD // ECH, ECH)


def kernel(x_movie, x_director, x_actor,
           ei_movie_director, ei_director_movie, ei_movie_actor, ei_actor_movie,
           W_proj_movie, b_proj_movie, W_proj_director, b_proj_director,
           W_proj_actor, b_proj_actor,
           att_src_movie_director, att_dst_movie_director,
           att_src_director_movie, att_dst_director_movie,
           att_src_movie_actor, att_dst_movie_actor,
           att_src_actor_movie, att_dst_actor_movie,
           q_sem, W_k, b_k, W_lin, b_lin):
    grid = R // BLK

    tab0, tab1, ad0, ad1 = pl.pallas_call(
        _tca_body,
        grid=(grid,),
        in_specs=[
            pl.BlockSpec((BLK, F_IN), lambda i: (i, 0)),
            pl.BlockSpec((BLK, F_IN), lambda i: (i, 0)),
            pl.BlockSpec((BLK, F_IN), lambda i: (i, 0)),
            pl.BlockSpec((F_IN, HID), lambda i: (0, 0)),
            pl.BlockSpec((1, HID), lambda i: (0, 0)),
            pl.BlockSpec((F_IN, HID), lambda i: (0, 0)),
            pl.BlockSpec((1, HID), lambda i: (0, 0)),
            pl.BlockSpec((F_IN, HID), lambda i: (0, 0)),
            pl.BlockSpec((1, HID), lambda i: (0, 0)),
            pl.BlockSpec((HID, 16), lambda i: (0, 0)),
            pl.BlockSpec((HID, 16), lambda i: (0, 0)),
            pl.BlockSpec((HID, 16), lambda i: (0, 0)),
            pl.BlockSpec((HID, 16), lambda i: (0, 0)),
        ],
        out_specs=[
            pl.BlockSpec((BLK, TW), lambda i: (i, 0)),
            pl.BlockSpec((BLK, TW), lambda i: (i, 0)),
            pl.BlockSpec((BLK, 16), lambda i: (i, 0)),
            pl.BlockSpec((BLK, 16), lambda i: (i, 0)),
        ],
        out_shape=[
            jax.ShapeDtypeStruct((R, TW), f32),
            jax.ShapeDtypeStruct((R, TW), f32),
            jax.ShapeDtypeStruct((R, 16), f32),
            jax.ShapeDtypeStruct((R, 16), f32),
        ],
    )(
        _pad_rows(x_director), _pad_rows(x_actor), _pad_rows(x_movie),
        W_proj_director, b_proj_director.reshape(1, HID),
        W_proj_actor, b_proj_actor.reshape(1, HID),
        W_proj_movie, b_proj_movie.reshape(1, HID),
        _blockdiag(att_src_director_movie), _blockdiag(att_src_actor_movie),
        _blockdiag(att_dst_director_movie), _blockdiag(att_dst_actor_movie),
    )

    ep0 = _pad_edges(ei_director_movie)
    ep1 = _pad_edges(ei_actor_movie)
    acc0, acc1 = _sc_call(tab0, tab1, ad0, ad1, ep0, ep1)

    rep_cols = (jnp.arange(HID, dtype=jnp.int32) // DH)
    rep = (rep_cols[None, :] == jnp.arange(HEADS, dtype=jnp.int32)[:, None]).astype(f32)
    agg0, agg1, cs0, cs1 = pl.pallas_call(
        _tcb1_body,
        grid=(grid,),
        in_specs=[
            pl.BlockSpec((BLK, TW), lambda i: (i, 0)),
            pl.BlockSpec((BLK, TW), lambda i: (i, 0)),
            pl.BlockSpec((HID, HID), lambda i: (0, 0)),
            pl.BlockSpec((1, HID), lambda i: (0, 0)),
            pl.BlockSpec((HEADS, HID), lambda i: (0, 0)),
        ],
        out_specs=[
            pl.BlockSpec((BLK, HID), lambda i: (i, 0)),
            pl.BlockSpec((BLK, HID), lambda i: (i, 0)),
            pl.BlockSpec((1, HID), lambda i: (0, 0)),
            pl.BlockSpec((1, HID), lambda i: (0, 0)),
        ],
        out_shape=[
            jax.ShapeDtypeStruct((R, HID), f32),
            jax.ShapeDtypeStruct((R, HID), f32),
            jax.ShapeDtypeStruct((1, HID), f32),
            jax.ShapeDtypeStruct((1, HID), f32),
        ],
        compiler_params=pltpu.CompilerParams(
            dimension_semantics=("arbitrary",)),
    )(acc0, acc1, W_k, b_k.reshape(1, HID), rep)

    out = pl.pallas_call(
        _tcb2_body,
        grid=(grid,),
        in_specs=[
            pl.BlockSpec((BLK, HID), lambda i: (i, 0)),
            pl.BlockSpec((BLK, HID), lambda i: (i, 0)),
            pl.BlockSpec((1, HID), lambda i: (0, 0)),
            pl.BlockSpec((1, HID), lambda i: (0, 0)),
            pl.BlockSpec((1, HID), lambda i: (0, 0)),
            pl.BlockSpec((HID, C), lambda i: (0, 0)),
            pl.BlockSpec((1, C), lambda i: (0, 0)),
        ],
        out_specs=pl.BlockSpec((BLK, C), lambda i: (i, 0)),
        out_shape=jax.ShapeDtypeStruct((R, C), f32),
    )(agg0, agg1, cs0, cs1, q_sem.reshape(1, HID), W_lin, b_lin.reshape(1, C))

    return out[:N]
